# baseline (speedup 1.0000x reference)
import jax
import jax.numpy as jnp
from jax.experimental import pallas as pl

PAD_ID = 0
EOS_ID = 2
GROUPS = 4
EXPANSION = 4
PENALTY = 0.5
TOPK = 10
ROWS_PER_BLOCK = 8
NEG_INF = float("-inf")


def _topk_lse_kernel(x_ref, v_ref, i_ref, l_ref):
    x = x_ref[...]
    iota = jax.lax.broadcasted_iota(jnp.int32, x.shape, 1)
    big = jnp.int32(2147483647)
    xw = x
    vals = []
    idxs = []
    for _ in range(TOPK):
        vk = jnp.max(xw, axis=1)
        ik = jnp.min(jnp.where(xw == vk[:, None], iota, big), axis=1)
        vals.append(vk)
        idxs.append(ik)
        xw = jnp.where(iota == ik[:, None], NEG_INF, xw)
    v = jnp.stack(vals, axis=1)
    m = v[:, 0]
    s = jnp.sum(jnp.exp(xw - m[:, None]), axis=1)
    s = s + jnp.sum(jnp.exp(v - m[:, None]), axis=1)
    l_ref[...] = (m + jnp.log(s))[:, None]
    v_ref[...] = v
    i_ref[...] = jnp.stack(idxs, axis=1)


def kernel(new_logits, scores, output_seq):
    B, BW, V = new_logits.shape
    S = output_seq.shape[2]
    gs = BW // GROUPS
    R = ROWS_PER_BLOCK
    nrows = B * BW

    x2d = new_logits.reshape(nrows, V)
    topv, topi, lse = pl.pallas_call(
        _topk_lse_kernel,
        grid=(nrows // R,),
        in_specs=[pl.BlockSpec((R, V), lambda i: (i, 0))],
        out_specs=[
            pl.BlockSpec((R, TOPK), lambda i: (i, 0)),
            pl.BlockSpec((R, TOPK), lambda i: (i, 0)),
            pl.BlockSpec((R, 1), lambda i: (i, 0)),
        ],
        out_shape=[
            jax.ShapeDtypeStruct((nrows, TOPK), jnp.float32),
            jax.ShapeDtypeStruct((nrows, TOPK), jnp.int32),
            jax.ShapeDtypeStruct((nrows, 1), jnp.float32),
        ],
    )(x2d)

    topv = topv.reshape(B, BW, TOPK)
    topi = topi.reshape(B, BW, TOPK)
    lse = lse.reshape(B, BW)

    last_tok = output_seq[:, :, -1]
    done = (last_tok == PAD_ID) | (last_tok == EOS_ID)

    lp_top = topv - lse[:, :, None]
    karange = jnp.arange(TOPK, dtype=jnp.int32)
    pad_lp = jnp.where(karange == 0, 0.0, NEG_INF)
    cand_lp = jnp.where(done[:, :, None], pad_lp[None, None, :], lp_top)
    cand_id = jnp.where(done[:, :, None], karange[None, None, :], topi)

    old_sum = jnp.sum(scores, axis=-1)
    old_len = jnp.sum(output_seq != PAD_ID, axis=-1).astype(jnp.int32)

    sel_tok = []
    sel_val = []
    sel_beam = []
    for g in range(GROUPS):
        ids = cand_id[:, g * gs:(g + 1) * gs]
        lp = cand_lp[:, g * gs:(g + 1) * gs]
        if g > 0:
            used = jnp.concatenate(sel_tok, axis=1)
            hit = (ids[:, :, :, None] == used[:, None, None, :]) & (
                used[:, None, None, :] != PAD_ID)
            lp = lp - PENALTY * jnp.sum(hit.astype(jnp.float32), axis=-1)
        tv, tix = jax.lax.top_k(lp, EXPANSION)
        ttok = jnp.take_along_axis(ids, tix, axis=-1)
        b_sum = old_sum[:, g * gs:(g + 1) * gs]
        b_len = old_len[:, g * gs:(g + 1) * gs]
        csum = b_sum[:, :, None] + tv
        clen = b_len[:, :, None] + (ttok != PAD_ID).astype(jnp.int32)
        bscore = csum / ((5.0 + clen.astype(jnp.float32)) / 6.0)
        flat_score = bscore.reshape(B, gs * EXPANSION)
        _, sidx = jax.lax.top_k(flat_score, gs)
        src_local = sidx // EXPANSION
        sel_tok.append(jnp.take_along_axis(ttok.reshape(B, gs * EXPANSION), sidx, axis=-1))
        sel_val.append(jnp.take_along_axis(tv.reshape(B, gs * EXPANSION), sidx, axis=-1))
        sel_beam.append(g * gs + src_local)

    tok_all = jnp.concatenate(sel_tok, axis=1)
    val_all = jnp.concatenate(sel_val, axis=1)
    beam_all = jnp.concatenate(sel_beam, axis=1)

    src_seq = jnp.take_along_axis(output_seq, beam_all[:, :, None], axis=1)
    sum_src = jnp.take_along_axis(old_sum, beam_all, axis=1)
    len_src = jnp.take_along_axis(old_len, beam_all, axis=1)

    done2 = (src_seq[:, :, -1] == EOS_ID) | (src_seq[:, :, -1] == PAD_ID)
    last_col = jnp.where(done2, PAD_ID, tok_all)
    new_out = jnp.concatenate([src_seq, last_col[:, :, None]], axis=-1)

    out_len = len_src + (last_col != PAD_ID).astype(jnp.int32)
    score_final = (sum_src + val_all) / ((5.0 + out_len.astype(jnp.float32)) / 6.0)

    order = jnp.argsort(-score_final, axis=-1)
    sorted_scores = jnp.take_along_axis(score_final, order, axis=-1)
    out_sorted = jnp.take_along_axis(new_out, order[:, :, None], axis=1)
    return (out_sorted, sorted_scores, out_len)


# speedup vs baseline: 1.0507x; 1.0507x over previous
import jax
import jax.numpy as jnp
from jax.experimental import pallas as pl

PAD_ID = 0
EOS_ID = 2
GROUPS = 4
EXPANSION = 4
PENALTY = 0.5
TOPK = 10
ROWS_PER_BLOCK = 8
NEG_INF = float("-inf")


def _topk_lse_kernel(x_ref, v_ref, i_ref, l_ref):
    x = x_ref[...]
    iota = jax.lax.broadcasted_iota(jnp.int32, x.shape, 1)
    big = jnp.int32(2147483647)
    xw = x
    vals = []
    idxs = []
    for _ in range(TOPK):
        vk = jnp.max(xw, axis=1)
        ik = jnp.min(jnp.where(xw == vk[:, None], iota, big), axis=1)
        vals.append(vk)
        idxs.append(ik)
        xw = jnp.where(iota == ik[:, None], NEG_INF, xw)
    m = vals[0]
    s = jnp.sum(jnp.exp(x - m[:, None]), axis=1)
    l_ref[...] = (m + jnp.log(s))[:, None]
    v_ref[...] = jnp.stack(vals, axis=1)
    i_ref[...] = jnp.stack(idxs, axis=1)


def kernel(new_logits, scores, output_seq):
    B, BW, V = new_logits.shape
    S = output_seq.shape[2]
    gs = BW // GROUPS
    R = ROWS_PER_BLOCK
    nrows = B * BW

    x2d = new_logits.reshape(nrows, V)
    topv, topi, lse = pl.pallas_call(
        _topk_lse_kernel,
        grid=(nrows // R,),
        in_specs=[pl.BlockSpec((R, V), lambda i: (i, 0))],
        out_specs=[
            pl.BlockSpec((R, TOPK), lambda i: (i, 0)),
            pl.BlockSpec((R, TOPK), lambda i: (i, 0)),
            pl.BlockSpec((R, 1), lambda i: (i, 0)),
        ],
        out_shape=[
            jax.ShapeDtypeStruct((nrows, TOPK), jnp.float32),
            jax.ShapeDtypeStruct((nrows, TOPK), jnp.int32),
            jax.ShapeDtypeStruct((nrows, 1), jnp.float32),
        ],
    )(x2d)

    topv = topv.reshape(B, BW, TOPK)
    topi = topi.reshape(B, BW, TOPK)
    lse = lse.reshape(B, BW)

    last_tok = output_seq[:, :, -1]
    done = (last_tok == PAD_ID) | (last_tok == EOS_ID)

    lp_top = topv - lse[:, :, None]
    karange = jnp.arange(TOPK, dtype=jnp.int32)
    pad_lp = jnp.where(karange == 0, 0.0, NEG_INF)
    cand_lp = jnp.where(done[:, :, None], pad_lp[None, None, :], lp_top)
    cand_id = jnp.where(done[:, :, None], karange[None, None, :], topi)

    old_sum = jnp.sum(scores, axis=-1)
    old_len = jnp.sum(output_seq != PAD_ID, axis=-1).astype(jnp.int32)

    sel_tok = []
    sel_val = []
    sel_beam = []
    for g in range(GROUPS):
        ids = cand_id[:, g * gs:(g + 1) * gs]
        lp = cand_lp[:, g * gs:(g + 1) * gs]
        if g > 0:
            used = jnp.concatenate(sel_tok, axis=1)
            hit = (ids[:, :, :, None] == used[:, None, None, :]) & (
                used[:, None, None, :] != PAD_ID)
            lp = lp - PENALTY * jnp.sum(hit.astype(jnp.float32), axis=-1)
        tv, tix = jax.lax.top_k(lp, EXPANSION)
        ttok = jnp.take_along_axis(ids, tix, axis=-1)
        b_sum = old_sum[:, g * gs:(g + 1) * gs]
        b_len = old_len[:, g * gs:(g + 1) * gs]
        csum = b_sum[:, :, None] + tv
        clen = b_len[:, :, None] + (ttok != PAD_ID).astype(jnp.int32)
        bscore = csum / ((5.0 + clen.astype(jnp.float32)) / 6.0)
        flat_score = bscore.reshape(B, gs * EXPANSION)
        _, sidx = jax.lax.top_k(flat_score, gs)
        src_local = sidx // EXPANSION
        sel_tok.append(jnp.take_along_axis(ttok.reshape(B, gs * EXPANSION), sidx, axis=-1))
        sel_val.append(jnp.take_along_axis(tv.reshape(B, gs * EXPANSION), sidx, axis=-1))
        sel_beam.append(g * gs + src_local)

    tok_all = jnp.concatenate(sel_tok, axis=1)
    val_all = jnp.concatenate(sel_val, axis=1)
    beam_all = jnp.concatenate(sel_beam, axis=1)

    src_seq = jnp.take_along_axis(output_seq, beam_all[:, :, None], axis=1)
    sum_src = jnp.take_along_axis(old_sum, beam_all, axis=1)
    len_src = jnp.take_along_axis(old_len, beam_all, axis=1)

    done2 = (src_seq[:, :, -1] == EOS_ID) | (src_seq[:, :, -1] == PAD_ID)
    last_col = jnp.where(done2, PAD_ID, tok_all)
    new_out = jnp.concatenate([src_seq, last_col[:, :, None]], axis=-1)

    out_len = len_src + (last_col != PAD_ID).astype(jnp.int32)
    score_final = (sum_src + val_all) / ((5.0 + out_len.astype(jnp.float32)) / 6.0)

    order = jnp.argsort(-score_final, axis=-1)
    sorted_scores = jnp.take_along_axis(score_final, order, axis=-1)
    out_sorted = jnp.take_along_axis(new_out, order[:, :, None], axis=1)
    return (out_sorted, sorted_scores, out_len)
